# baseline (speedup 1.0000x reference)
import functools

import jax
import jax.numpy as jnp
from jax import lax
from jax.experimental import pallas as pl
from jax.experimental.pallas import tpu as pltpu
from jax.experimental.pallas import tpu_sc as plsc

_NSLOT = 8


def _make_sc_fetch(V, D, B):
    info = plsc.get_sparse_core_info()
    NC, NS = info.num_cores, info.num_subcores
    NW = NC * NS
    b_per_w = B // NW
    b_half = b_per_w // 2
    n_waves = b_half // _NSLOT
    t_max = (V - 1) // 128
    mesh = plsc.VectorSubcoreMesh(core_axis_name="c", subcore_axis_name="s")

    @functools.partial(
        pl.kernel,
        mesh=mesh,
        out_type=jax.ShapeDtypeStruct((D, B), jnp.float32),
        scratch_types=[
            pltpu.VMEM((b_per_w + 32,), jnp.int32),
            pltpu.VMEM((_NSLOT, D, 128), jnp.float32),
            pltpu.VMEM((D, b_half), jnp.float32),
            [pltpu.SemaphoreType.DMA] * _NSLOT,
            pltpu.SemaphoreType.DMA,
        ],
        compiler_params=pltpu.CompilerParams(use_tc_tiling_on_sc=True,
                                             needs_layout_passes=False),
    )
    def fetch(tableT_hbm, idx_hbm, out_hbm, idx_v, blk_v, rows_v, sems,
              wsem):
        wid = lax.axis_index("s") * NC + lax.axis_index("c")
        base = wid * b_per_w
        pltpu.sync_copy(idx_hbm.at[pl.ds(base, b_per_w)],
                        idx_v.at[pl.ds(8, b_per_w)])
        iota16 = jax.lax.broadcasted_iota(jnp.int32, (16,), 0)
        lo8 = iota16 < 8
        slot16 = jax.lax.bitwise_and(iota16, 7)
        feats = [jnp.where(lo8, 2 * k, 2 * k + 1) for k in range(D // 2)]

        def issue(j, slot):
            r = idx_v[pl.ds(j + 8, 16)][0]
            t = jnp.minimum(lax.shift_right_logical(r, 7), t_max)
            pltpu.make_async_copy(
                tableT_hbm.at[:, pl.ds(pl.multiple_of(t * 128, 128), 128)],
                blk_v.at[slot],
                sems[slot],
            ).start()

        def extract8(j0, j_rel0):
            ca = idx_v[pl.ds(j0 + 8, 16)]
            cb = idx_v[pl.ds(j0, 16)]
            cols = lax.bitwise_and(jnp.where(lo8, ca, cb), 127)
            jcol = slot16 + j_rel0
            for k in range(D // 2):
                vals = plsc.load_gather(blk_v, [slot16, feats[k], cols])
                plsc.store_scatter(rows_v, [feats[k], jcol], vals)

        for phase in range(2):
            off = phase * b_half

            for slot in range(_NSLOT):
                issue(off + slot, slot)

            def wave(w, carry):
                j_rel0 = w * _NSLOT
                j0 = off + j_rel0
                for slot in range(_NSLOT):
                    pltpu.make_async_copy(
                        tableT_hbm.at[:, pl.ds(0, 128)], blk_v.at[slot],
                        sems[slot],
                    ).wait()
                extract8(j0, j_rel0)

                @pl.when(w + 1 < n_waves)
                def _():
                    for slot in range(_NSLOT):
                        issue(j0 + slot + _NSLOT, slot)
                return carry

            lax.fori_loop(0, n_waves, wave, 0)
            wcopies = [
                pltpu.make_async_copy(
                    rows_v.at[c], out_hbm.at[c, pl.ds(base + off, b_half)],
                    wsem)
                for c in range(D)
            ]
            for cp in wcopies:
                cp.start()
            for cp in wcopies:
                cp.wait()

    return fetch


def _mlp_body(embT_ref, genT_ref, w1aT_ref, w1bT_ref, b1_ref, w2T_ref,
              b2_ref, outT_ref):
    hT = jnp.dot(w1aT_ref[...], embT_ref[...],
                 preferred_element_type=jnp.float32)
    hT = hT + jnp.dot(w1bT_ref[...], genT_ref[...],
                      preferred_element_type=jnp.float32)
    hT = jnp.maximum(hT + b1_ref[...], 0.0)
    yT = jnp.dot(w2T_ref[...], hT, preferred_element_type=jnp.float32)
    yT = yT + b2_ref[...]
    norm = jnp.sqrt(jnp.sum(yT * yT, axis=0, keepdims=True))
    outT_ref[...] = yT / jnp.maximum(norm, 1e-12)


def kernel(item_ids, genre_vectors, table, W1, b1, W2, b2):
    B, = item_ids.shape
    V, D = table.shape
    G = genre_vectors.shape[1]
    H = W1.shape[1]

    embT = _make_sc_fetch(V, D, B)(table.T, item_ids.astype(jnp.int32))

    genT = genre_vectors.T
    w1aT = W1[:D].T
    w1bT = W1[D:].T
    w2T = W2.T
    b1c = b1.reshape(H, 1)
    b2c = b2.reshape(D, 1)

    BB = 2048
    grid = (B // BB,)
    outT = pl.pallas_call(
        _mlp_body,
        grid=grid,
        in_specs=[
            pl.BlockSpec((D, BB), lambda i: (0, i)),
            pl.BlockSpec((G, BB), lambda i: (0, i)),
            pl.BlockSpec((H, D), lambda i: (0, 0)),
            pl.BlockSpec((H, G), lambda i: (0, 0)),
            pl.BlockSpec((H, 1), lambda i: (0, 0)),
            pl.BlockSpec((D, H), lambda i: (0, 0)),
            pl.BlockSpec((D, 1), lambda i: (0, 0)),
        ],
        out_specs=pl.BlockSpec((D, BB), lambda i: (0, i)),
        out_shape=jax.ShapeDtypeStruct((D, B), jnp.float32),
    )(embT, genT, w1aT, w1bT, b1c, w2T, b2c)
    return outT.T


# speedup vs baseline: 1.2483x; 1.2483x over previous
import functools

import jax
import jax.numpy as jnp
from jax import lax
from jax.experimental import pallas as pl
from jax.experimental.pallas import tpu as pltpu
from jax.experimental.pallas import tpu_sc as plsc

_NSLOT = 8


def _make_sc_fetch(V, D, B):
    info = plsc.get_sparse_core_info()
    NC, NS = info.num_cores, info.num_subcores
    NW = NC * NS
    b_per_w = B // NW
    b_half = b_per_w // 2
    n_waves = b_half // _NSLOT
    t_max = (V - 1) // 128
    mesh = plsc.VectorSubcoreMesh(core_axis_name="c", subcore_axis_name="s")

    @functools.partial(
        pl.kernel,
        mesh=mesh,
        out_type=jax.ShapeDtypeStruct((D, B), jnp.float32),
        scratch_types=[
            pltpu.VMEM((b_per_w + 16,), jnp.int32),
            pltpu.VMEM((_NSLOT, D, 128), jnp.float32),
            pltpu.VMEM((D, b_half), jnp.float32),
            [pltpu.SemaphoreType.DMA] * _NSLOT,
            pltpu.SemaphoreType.DMA,
        ],
        compiler_params=pltpu.CompilerParams(use_tc_tiling_on_sc=True,
                                             needs_layout_passes=False),
    )
    def fetch(tableT_hbm, idx_hbm, out_hbm, idx_v, blk_v, rows_v, sems,
              wsem):
        wid = lax.axis_index("s") * NC + lax.axis_index("c")
        base = wid * b_per_w
        pltpu.sync_copy(idx_hbm.at[pl.ds(base, b_per_w)],
                        idx_v.at[pl.ds(0, b_per_w)])
        rows16 = [jax.lax.broadcasted_iota(jnp.int32, (16,), 0) + 16 * k
                  for k in range(D // 16)]

        def issue(j, slot):
            r = idx_v[pl.ds(j, 16)][0]
            t = jnp.minimum(lax.shift_right_logical(r, 7), t_max)
            pltpu.make_async_copy(
                tableT_hbm.at[:, pl.ds(pl.multiple_of(t * 128, 128), 128)],
                blk_v.at[slot],
                sems[slot],
            ).start()

        def extract(j, j_rel, slot):
            r = idx_v[pl.ds(j, 16)][0]
            col = jnp.broadcast_to(lax.bitwise_and(r, 127), (16,))
            jcol = jnp.broadcast_to(j_rel, (16,))
            for k in range(D // 16):
                vals = plsc.load_gather(blk_v.at[slot], [rows16[k], col])
                plsc.store_scatter(rows_v, [rows16[k], jcol], vals)

        for phase in range(2):
            off = phase * b_half

            for slot in range(_NSLOT):
                issue(off + slot, slot)

            def wave(w, carry):
                for slot in range(_NSLOT):
                    j_rel = w * _NSLOT + slot
                    j = off + j_rel
                    pltpu.make_async_copy(
                        tableT_hbm.at[:, pl.ds(0, 128)], blk_v.at[slot],
                        sems[slot],
                    ).wait()
                    extract(j, j_rel, slot)

                    @pl.when(w + 1 < n_waves)
                    def _():
                        issue(j + _NSLOT, slot)
                return carry

            lax.fori_loop(0, n_waves, wave, 0)
            wcopies = [
                pltpu.make_async_copy(
                    rows_v.at[c], out_hbm.at[c, pl.ds(base + off, b_half)],
                    wsem)
                for c in range(D)
            ]
            for cp in wcopies:
                cp.start()
            for cp in wcopies:
                cp.wait()

    return fetch


def _mlp_body(embT_ref, genT_ref, w1aT_ref, w1bT_ref, b1_ref, w2T_ref,
              b2_ref, outT_ref):
    hT = jnp.dot(w1aT_ref[...], embT_ref[...],
                 preferred_element_type=jnp.float32)
    hT = hT + jnp.dot(w1bT_ref[...], genT_ref[...],
                      preferred_element_type=jnp.float32)
    hT = jnp.maximum(hT + b1_ref[...], 0.0)
    yT = jnp.dot(w2T_ref[...], hT, preferred_element_type=jnp.float32)
    yT = yT + b2_ref[...]
    norm = jnp.sqrt(jnp.sum(yT * yT, axis=0, keepdims=True))
    outT_ref[...] = yT / jnp.maximum(norm, 1e-12)


def kernel(item_ids, genre_vectors, table, W1, b1, W2, b2):
    B, = item_ids.shape
    V, D = table.shape
    G = genre_vectors.shape[1]
    H = W1.shape[1]

    embT = _make_sc_fetch(V, D, B)(table.T, item_ids.astype(jnp.int32))

    genT = genre_vectors.T
    w1aT = W1[:D].T
    w1bT = W1[D:].T
    w2T = W2.T
    b1c = b1.reshape(H, 1)
    b2c = b2.reshape(D, 1)

    BB = 2048
    grid = (B // BB,)
    outT = pl.pallas_call(
        _mlp_body,
        grid=grid,
        in_specs=[
            pl.BlockSpec((D, BB), lambda i: (0, i)),
            pl.BlockSpec((G, BB), lambda i: (0, i)),
            pl.BlockSpec((H, D), lambda i: (0, 0)),
            pl.BlockSpec((H, G), lambda i: (0, 0)),
            pl.BlockSpec((H, 1), lambda i: (0, 0)),
            pl.BlockSpec((D, H), lambda i: (0, 0)),
            pl.BlockSpec((D, 1), lambda i: (0, 0)),
        ],
        out_specs=pl.BlockSpec((D, BB), lambda i: (0, i)),
        out_shape=jax.ShapeDtypeStruct((D, B), jnp.float32),
    )(embT, genT, w1aT, w1bT, b1c, w2T, b2c)
    return outT.T
